# Initial kernel scaffold; baseline (speedup 1.0000x reference)
import jax
import jax.numpy as jnp
from jax.experimental import pallas as pl


def kernel(node_features, edge_index, cycle_mask, W1, b1, W2, b2, cycle_penalty, min_sum_scaler):
    raise NotImplementedError("write your pallas kernel here")


# speedup vs baseline: 123.5770x; 123.5770x over previous
import functools

import jax
import jax.numpy as jnp
from jax import lax
from jax.experimental import pallas as pl
from jax.experimental.pallas import tpu as pltpu
from jax.experimental.pallas import tpu_sc as plsc

N_NODES = 100000
N_EDGES = 3200000
NUM_HEADS = 4
NC = 2
NS = 16
NW = NC * NS
LANES = 16
EB = 1024
NBLOCKS = N_EDGES // EB
MAXT = (NBLOCKS + NW - 1) // NW
ROWS_A = 6256
ROWS_LAST = N_NODES - 15 * ROWS_A

_mesh = plsc.VectorSubcoreMesh(
    core_axis_name="c", subcore_axis_name="s", num_cores=NC, num_subcores=NS)


def _edge_pass_body(ei_hbm, cm_hbm, nf_hbm, par_hbm, zer_hbm, sts_hbm,
                    par_v, src_v, dst_v, cm_v, xs_v, xd_v, upd_v,
                    nfb_v, mrg_v, scr_v, nf_sh, scr_sh, st_sh):
    cid = lax.axis_index("c")
    sid = lax.axis_index("s")
    wid = sid * NC + cid

    pltpu.sync_copy(par_hbm, par_v)

    @pl.when(sid == 0)
    def _():
        pltpu.sync_copy(nf_hbm, nf_sh)

    iota = lax.iota(jnp.int32, LANES)
    base_row = pl.multiple_of(sid * ROWS_A, 8)

    def mx_body(i, m):
        return jnp.maximum(m, jnp.abs(nfb_v[pl.ds(i * LANES, LANES)]))

    @pl.when(sid < NS - 1)
    def _():
        pltpu.sync_copy(zer_hbm, st_sh.at[pl.ds(base_row, ROWS_A)])
        pltpu.sync_copy(nf_hbm.at[pl.ds(base_row, ROWS_A)], nfb_v)
        m = lax.fori_loop(0, ROWS_A // LANES, mx_body,
                          jnp.zeros((LANES,), jnp.float32))
        mrg_v[...] = m

    @pl.when(sid == NS - 1)
    def _():
        pltpu.sync_copy(zer_hbm.at[pl.ds(0, ROWS_LAST)],
                        st_sh.at[pl.ds(base_row, ROWS_LAST)])
        pltpu.sync_copy(nf_hbm.at[pl.ds(base_row, ROWS_LAST)],
                        nfb_v.at[pl.ds(0, ROWS_LAST)])
        m = lax.fori_loop(0, ROWS_LAST // LANES, mx_body,
                          jnp.zeros((LANES,), jnp.float32))
        mrg_v[...] = m

    pltpu.sync_copy(mrg_v, scr_sh.at[sid])
    plsc.subcore_barrier()

    pltpu.sync_copy(scr_sh, scr_v)
    mx = scr_v[0]
    for i in range(1, NS):
        mx = jnp.maximum(mx, scr_v[i])
    for k in (1, 2, 4, 8):
        mrg_v[...] = mx
        mx = jnp.maximum(mx, plsc.load_gather(
            mrg_v, [jnp.bitwise_xor(iota, jnp.int32(k))]))

    A = [par_v[h] for h in range(NUM_HEADS)]
    C = [par_v[NUM_HEADS + h] for h in range(NUM_HEADS)]
    D = [par_v[2 * NUM_HEADS + h] for h in range(NUM_HEADS)]
    E0 = [par_v[3 * NUM_HEADS + h] for h in range(NUM_HEADS)]
    PEN = [par_v[4 * NUM_HEADS + h] for h in range(NUM_HEADS)]
    SH = [jnp.abs(A[h]) * mx + jnp.abs(C[h]) * mx + jnp.abs(D[h])
          + jnp.abs(E0[h]) + jnp.abs(PEN[h]) for h in range(NUM_HEADS)]

    def blk_body(t, carry):
        g = wid + t * NW

        @pl.when(g < NBLOCKS)
        def _():
            pltpu.sync_copy(ei_hbm.at[0, g], src_v)
            pltpu.sync_copy(ei_hbm.at[1, g], dst_v)
            pltpu.sync_copy(cm_hbm.at[g], cm_v)
            for rr in range(8):
                pltpu.sync_copy(nf_sh.at[src_v.at[rr]], xs_v.at[rr])
                pltpu.sync_copy(nf_sh.at[dst_v.at[rr]], xd_v.at[rr])

                def cc_body(ccj, cc_carry):
                    col = ccj * LANES
                    xs = xs_v[rr, pl.ds(col, LANES)]
                    xd = xd_v[rr, pl.ds(col, LANES)]
                    cmv = cm_v[rr, pl.ds(col, LANES)]
                    rowv = rr * 128 + col + iota
                    for h in range(NUM_HEADS):
                        r = A[h] * xs + C[h] * xd + D[h] * cmv + E0[h]
                        r = jnp.maximum(r, 0.2 * r)
                        ex = jnp.exp(r + cmv * PEN[h] - SH[h])
                        plsc.store_scatter(
                            upd_v, [rowv, jnp.full((LANES,), h, jnp.int32)], ex)
                        plsc.store_scatter(
                            upd_v,
                            [rowv, jnp.full((LANES,), NUM_HEADS + h, jnp.int32)],
                            xs * ex)
                    return cc_carry

                lax.fori_loop(0, 128 // LANES, cc_body, 0)
                pltpu.sync_copy(upd_v.at[pl.ds(rr * 128, 128)],
                                st_sh.at[dst_v.at[rr]], add=True)
        return carry

    lax.fori_loop(0, MAXT, blk_body, 0)

    plsc.subcore_barrier()

    @pl.when(sid < NS - 1)
    def _():
        pltpu.sync_copy(st_sh.at[pl.ds(base_row, ROWS_A)],
                        sts_hbm.at[cid, pl.ds(base_row, ROWS_A)])

    @pl.when(sid == NS - 1)
    def _():
        pltpu.sync_copy(st_sh.at[pl.ds(base_row, ROWS_LAST)],
                        sts_hbm.at[cid, pl.ds(base_row, ROWS_LAST)])


_edge_pass = functools.partial(
    pl.kernel,
    out_type=jax.ShapeDtypeStruct((NC, N_NODES, 2 * NUM_HEADS), jnp.float32),
    mesh=_mesh,
    compiler_params=pltpu.CompilerParams(needs_layout_passes=False,
                                         use_tc_tiling_on_sc=False),
    scratch_types=[
        pltpu.VMEM((5 * NUM_HEADS, LANES), jnp.float32),
        pltpu.VMEM((8, 128), jnp.int32),
        pltpu.VMEM((8, 128), jnp.int32),
        pltpu.VMEM((8, 128), jnp.float32),
        pltpu.VMEM((8, 128), jnp.float32),
        pltpu.VMEM((8, 128), jnp.float32),
        pltpu.VMEM((EB, 2 * NUM_HEADS), jnp.float32),
        pltpu.VMEM((ROWS_A,), jnp.float32),
        pltpu.VMEM((LANES,), jnp.float32),
        pltpu.VMEM((NS, LANES), jnp.float32),
        pltpu.VMEM_SHARED((N_NODES,), jnp.float32),
        pltpu.VMEM_SHARED((NS, LANES), jnp.float32),
        pltpu.VMEM_SHARED((N_NODES, 2 * NUM_HEADS), jnp.float32),
    ],
)(_edge_pass_body)


_PADN = 100352
_KB = 512
_GRID = _PADN // _KB


def _combine_body(scal_ref, st_ref, out_ref):
    x = st_ref[...]
    s = x[0] + x[1]
    S = s[0:NUM_HEADS]
    T = s[NUM_HEADS:]
    q = jnp.where(S > 0.0, T / S, 0.0)
    out_ref[...] = (jnp.sum(q, axis=0) * scal_ref[0, 0])[None, :]


_combine = pl.pallas_call(
    _combine_body,
    grid=(_GRID,),
    in_specs=[
        pl.BlockSpec(memory_space=pltpu.SMEM),
        pl.BlockSpec((NC, 2 * NUM_HEADS, _KB), lambda i: (0, 0, i)),
    ],
    out_specs=pl.BlockSpec((1, _KB), lambda i: (0, i)),
    out_shape=jax.ShapeDtypeStruct((1, _PADN), jnp.float32),
)


def kernel(node_features, edge_index, cycle_mask, W1, b1, W2, b2,
           cycle_penalty, min_sum_scaler):
    w1 = W1[:, 0]
    a = W2[:, :16] @ w1
    c = W2[:, 16:32] @ w1
    d = W2[:, 32]
    e = b2 + (W2[:, :16] + W2[:, 16:32]) @ b1
    par = jnp.concatenate([a, c, d, e, cycle_penalty])
    par16 = jnp.broadcast_to(par[:, None], (5 * NUM_HEADS, LANES))

    ei = edge_index.reshape(2, NBLOCKS, 8, 128)
    cmr = cycle_mask.reshape(NBLOCKS, 8, 128)
    zer = jnp.zeros((ROWS_A, 2 * NUM_HEADS), jnp.float32)

    sts = _edge_pass(ei, cmr, node_features, par16, zer)

    stt = jnp.pad(jnp.transpose(sts, (0, 2, 1)),
                  ((0, 0), (0, 0), (0, _PADN - N_NODES)))
    scal = (min_sum_scaler * (1.0 / NUM_HEADS)).reshape(1, 1)
    out = _combine(scal, stt)
    return out.reshape(_PADN)[:N_NODES]


# speedup vs baseline: 210.7700x; 1.7056x over previous
import functools

import jax
import jax.numpy as jnp
from jax import lax
from jax.experimental import pallas as pl
from jax.experimental.pallas import tpu as pltpu
from jax.experimental.pallas import tpu_sc as plsc

N_NODES = 100000
N_EDGES = 3200000
NUM_HEADS = 4
NC = 2
NS = 16
NW = NC * NS
LANES = 16
EB = 1024
NBLOCKS = N_EDGES // EB
MAXT = (NBLOCKS + NW - 1) // NW
ROWS_A = 6256
ROWS_LAST = N_NODES - 15 * ROWS_A

_mesh = plsc.VectorSubcoreMesh(
    core_axis_name="c", subcore_axis_name="s", num_cores=NC, num_subcores=NS)


def _edge_pass_body(ei_hbm, cm_hbm, nf_hbm, par_hbm, zer_hbm, sts_hbm,
                    par_v, src_v0, dst_v0, cm_v0, xs_v0, xd_v0, upd_v0,
                    src_v1, dst_v1, cm_v1, xs_v1, xd_v1, upd_v1,
                    nfb_v, mrg_v, scr_v, nf_sh, scr_sh, st_sh,
                    sem_in0, sem_in1, sem_g0, sem_g1, sem_s0, sem_s1):
    src_v = [src_v0, src_v1]
    dst_v = [dst_v0, dst_v1]
    cm_v = [cm_v0, cm_v1]
    xs_v = [xs_v0, xs_v1]
    xd_v = [xd_v0, xd_v1]
    upd_v = [upd_v0, upd_v1]
    sem_in = [sem_in0, sem_in1]
    sem_g = [sem_g0, sem_g1]
    sem_s = [sem_s0, sem_s1]
    cid = lax.axis_index("c")
    sid = lax.axis_index("s")
    wid = sid * NC + cid

    pltpu.sync_copy(par_hbm, par_v)

    @pl.when(sid == 0)
    def _():
        pltpu.sync_copy(nf_hbm, nf_sh)

    iota = lax.iota(jnp.int32, LANES)
    base_row = pl.multiple_of(sid * ROWS_A, 8)

    def mx_body(i, m):
        return jnp.maximum(m, jnp.abs(nfb_v[pl.ds(i * LANES, LANES)]))

    @pl.when(sid < NS - 1)
    def _():
        pltpu.sync_copy(zer_hbm, st_sh.at[pl.ds(base_row, ROWS_A)])
        pltpu.sync_copy(nf_hbm.at[pl.ds(base_row, ROWS_A)], nfb_v)
        m = lax.fori_loop(0, ROWS_A // LANES, mx_body,
                          jnp.zeros((LANES,), jnp.float32))
        mrg_v[...] = m

    @pl.when(sid == NS - 1)
    def _():
        pltpu.sync_copy(zer_hbm.at[pl.ds(0, ROWS_LAST)],
                        st_sh.at[pl.ds(base_row, ROWS_LAST)])
        pltpu.sync_copy(nf_hbm.at[pl.ds(base_row, ROWS_LAST)],
                        nfb_v.at[pl.ds(0, ROWS_LAST)])
        m = lax.fori_loop(0, ROWS_LAST // LANES, mx_body,
                          jnp.zeros((LANES,), jnp.float32))
        mrg_v[...] = m

    pltpu.sync_copy(mrg_v, scr_sh.at[sid])
    plsc.subcore_barrier()

    pltpu.sync_copy(scr_sh, scr_v)
    mx = scr_v[0]
    for i in range(1, NS):
        mx = jnp.maximum(mx, scr_v[i])
    for k in (1, 2, 4, 8):
        mrg_v[...] = mx
        mx = jnp.maximum(mx, plsc.load_gather(
            mrg_v, [jnp.bitwise_xor(iota, jnp.int32(k))]))

    A = [par_v[h] for h in range(NUM_HEADS)]
    C = [par_v[NUM_HEADS + h] for h in range(NUM_HEADS)]
    D = [par_v[2 * NUM_HEADS + h] for h in range(NUM_HEADS)]
    E0 = [par_v[3 * NUM_HEADS + h] for h in range(NUM_HEADS)]
    PEN = [par_v[4 * NUM_HEADS + h] for h in range(NUM_HEADS)]
    SH = [jnp.abs(A[h]) * mx + jnp.abs(C[h]) * mx + jnp.abs(D[h])
          + jnp.abs(E0[h]) + jnp.abs(PEN[h]) for h in range(NUM_HEADS)]

    def _fire_loads(par, g):
        pltpu.async_copy(ei_hbm.at[0, g], src_v[par], sem_in[par])
        pltpu.async_copy(ei_hbm.at[1, g], dst_v[par], sem_in[par])
        pltpu.async_copy(cm_hbm.at[g], cm_v[par], sem_in[par])

    def _wait_loads(par, g):
        pltpu.make_async_copy(ei_hbm.at[0, g], src_v[par], sem_in[par]).wait()
        pltpu.make_async_copy(ei_hbm.at[1, g], dst_v[par], sem_in[par]).wait()
        pltpu.make_async_copy(cm_hbm.at[g], cm_v[par], sem_in[par]).wait()

    def _fire_gathers(par):
        for rr in range(8):
            pltpu.async_copy(nf_sh.at[src_v[par].at[rr]], xs_v[par].at[rr],
                             sem_g[par])
            pltpu.async_copy(nf_sh.at[dst_v[par].at[rr]], xd_v[par].at[rr],
                             sem_g[par])

    def _wait_gathers(par):
        for rr in range(8):
            pltpu.make_async_copy(nf_sh.at[src_v[par].at[rr]],
                                  xs_v[par].at[rr], sem_g[par]).wait()
            pltpu.make_async_copy(nf_sh.at[dst_v[par].at[rr]],
                                  xd_v[par].at[rr], sem_g[par]).wait()

    def _fire_scatters(par):
        for rr in range(8):
            pltpu.async_copy(upd_v[par].at[pl.ds(rr * 128, 128)],
                             st_sh.at[dst_v[par].at[rr]], sem_s[par], add=True)

    def _wait_scatters(par):
        for rr in range(8):
            pltpu.make_async_copy(upd_v[par].at[pl.ds(rr * 128, 128)],
                                  st_sh.at[dst_v[par].at[rr]],
                                  sem_s[par]).wait()

    def _compute(par):
        for rr in range(8):
            def cc_body(ccj, cc_carry, rr=rr):
                col = ccj * LANES
                xs = xs_v[par][rr, pl.ds(col, LANES)]
                xd = xd_v[par][rr, pl.ds(col, LANES)]
                cmv = cm_v[par][rr, pl.ds(col, LANES)]
                rowv = rr * 128 + col + iota
                for h in range(NUM_HEADS):
                    r = A[h] * xs + C[h] * xd + D[h] * cmv + E0[h]
                    r = jnp.maximum(r, 0.2 * r)
                    ex = jnp.exp(r + cmv * PEN[h] - SH[h])
                    plsc.store_scatter(
                        upd_v[par],
                        [rowv, jnp.full((LANES,), h, jnp.int32)], ex)
                    plsc.store_scatter(
                        upd_v[par],
                        [rowv, jnp.full((LANES,), NUM_HEADS + h, jnp.int32)],
                        xs * ex)
                return cc_carry

            lax.fori_loop(0, 128 // LANES, cc_body, 0)

    def blk2_body(u, carry):
        for par in range(2):
            t2 = 2 * u + par
            g = wid + t2 * NW
            gprev = g - NW
            gnext = g + NW

            @pl.when(g < NBLOCKS)
            def _(par=par, g=g):
                _wait_loads(par, g)
                _fire_gathers(par)

            @pl.when(jnp.logical_and(gprev >= 0, gprev < NBLOCKS))
            def _(par=par):
                _wait_scatters(1 - par)

            @pl.when(gnext < NBLOCKS)
            def _(par=par, gnext=gnext):
                _fire_loads(1 - par, gnext)

            @pl.when(g < NBLOCKS)
            def _(par=par):
                _wait_gathers(par)
                _compute(par)
                _fire_scatters(par)
        return carry

    _fire_loads(0, wid)
    lax.fori_loop(0, MAXT // 2, blk2_body, 0)

    g_last = wid + (MAXT - 1) * NW

    @pl.when(g_last < NBLOCKS)
    def _():
        _wait_scatters((MAXT - 1) & 1)

    plsc.subcore_barrier()

    @pl.when(sid < NS - 1)
    def _():
        pltpu.sync_copy(st_sh.at[pl.ds(base_row, ROWS_A)],
                        sts_hbm.at[cid, pl.ds(base_row, ROWS_A)])

    @pl.when(sid == NS - 1)
    def _():
        pltpu.sync_copy(st_sh.at[pl.ds(base_row, ROWS_LAST)],
                        sts_hbm.at[cid, pl.ds(base_row, ROWS_LAST)])


_edge_pass = functools.partial(
    pl.kernel,
    out_type=jax.ShapeDtypeStruct((NC, N_NODES, 2 * NUM_HEADS), jnp.float32),
    mesh=_mesh,
    compiler_params=pltpu.CompilerParams(needs_layout_passes=False,
                                         use_tc_tiling_on_sc=False),
    scratch_types=[
        pltpu.VMEM((5 * NUM_HEADS, LANES), jnp.float32),
    ] + 2 * [
        pltpu.VMEM((8, 128), jnp.int32),
        pltpu.VMEM((8, 128), jnp.int32),
        pltpu.VMEM((8, 128), jnp.float32),
        pltpu.VMEM((8, 128), jnp.float32),
        pltpu.VMEM((8, 128), jnp.float32),
        pltpu.VMEM((EB, 2 * NUM_HEADS), jnp.float32),
    ] + [
        pltpu.VMEM((ROWS_A,), jnp.float32),
        pltpu.VMEM((LANES,), jnp.float32),
        pltpu.VMEM((NS, LANES), jnp.float32),
        pltpu.VMEM_SHARED((N_NODES,), jnp.float32),
        pltpu.VMEM_SHARED((NS, LANES), jnp.float32),
        pltpu.VMEM_SHARED((N_NODES, 2 * NUM_HEADS), jnp.float32),
    ] + 6 * [pltpu.SemaphoreType.DMA],
)(_edge_pass_body)


_PADN = 100352
_KB = 512
_GRID = _PADN // _KB


def _combine_body(scal_ref, st_ref, out_ref):
    x = st_ref[...]
    s = x[0] + x[1]
    S = s[0:NUM_HEADS]
    T = s[NUM_HEADS:]
    q = jnp.where(S > 0.0, T / S, 0.0)
    out_ref[...] = (jnp.sum(q, axis=0) * scal_ref[0, 0])[None, :]


_combine = pl.pallas_call(
    _combine_body,
    grid=(_GRID,),
    in_specs=[
        pl.BlockSpec(memory_space=pltpu.SMEM),
        pl.BlockSpec((NC, 2 * NUM_HEADS, _KB), lambda i: (0, 0, i)),
    ],
    out_specs=pl.BlockSpec((1, _KB), lambda i: (0, i)),
    out_shape=jax.ShapeDtypeStruct((1, _PADN), jnp.float32),
)


def kernel(node_features, edge_index, cycle_mask, W1, b1, W2, b2,
           cycle_penalty, min_sum_scaler):
    w1 = W1[:, 0]
    a = W2[:, :16] @ w1
    c = W2[:, 16:32] @ w1
    d = W2[:, 32]
    e = b2 + (W2[:, :16] + W2[:, 16:32]) @ b1
    par = jnp.concatenate([a, c, d, e, cycle_penalty])
    par16 = jnp.broadcast_to(par[:, None], (5 * NUM_HEADS, LANES))

    ei = edge_index.reshape(2, NBLOCKS, 8, 128)
    cmr = cycle_mask.reshape(NBLOCKS, 8, 128)
    zer = jnp.zeros((ROWS_A, 2 * NUM_HEADS), jnp.float32)

    sts = _edge_pass(ei, cmr, node_features, par16, zer)

    stt = jnp.pad(jnp.transpose(sts, (0, 2, 1)),
                  ((0, 0), (0, 0), (0, _PADN - N_NODES)))
    scal = (min_sum_scaler * (1.0 / NUM_HEADS)).reshape(1, 1)
    out = _combine(scal, stt)
    return out.reshape(_PADN)[:N_NODES]


# speedup vs baseline: 219.2195x; 1.0401x over previous
import functools

import jax
import jax.numpy as jnp
from jax import lax
from jax.experimental import pallas as pl
from jax.experimental.pallas import tpu as pltpu
from jax.experimental.pallas import tpu_sc as plsc

N_NODES = 100000
N_EDGES = 3200000
NUM_HEADS = 4
NC = 2
NS = 16
NW = NC * NS
LANES = 16
EB = 1024
NBLOCKS = N_EDGES // EB
MAXT = (NBLOCKS + NW - 1) // NW
ROWS_A = 6256
ROWS_LAST = N_NODES - 15 * ROWS_A

_mesh = plsc.VectorSubcoreMesh(
    core_axis_name="c", subcore_axis_name="s", num_cores=NC, num_subcores=NS)


def _edge_pass_body(ei_hbm, cm_hbm, nf_hbm, par_hbm, zer_hbm, sts_hbm,
                    par_v, src_v0, dst_v0, cm_v0, xs_v0, xd_v0, upd_v0,
                    src_v1, dst_v1, cm_v1, xs_v1, xd_v1, upd_v1,
                    nfb_v, mrg_v, scr_v, nf_sh, scr_sh, st_sh,
                    sem_in0, sem_in1, sem_g0, sem_g1, sem_s0, sem_s1):
    src_v = [src_v0, src_v1]
    dst_v = [dst_v0, dst_v1]
    cm_v = [cm_v0, cm_v1]
    xs_v = [xs_v0, xs_v1]
    xd_v = [xd_v0, xd_v1]
    upd_v = [upd_v0, upd_v1]
    sem_in = [sem_in0, sem_in1]
    sem_g = [sem_g0, sem_g1]
    sem_s = [sem_s0, sem_s1]
    cid = lax.axis_index("c")
    sid = lax.axis_index("s")
    wid = sid * NC + cid

    pltpu.sync_copy(par_hbm, par_v)

    @pl.when(sid == 0)
    def _():
        pltpu.sync_copy(nf_hbm, nf_sh)

    iota = lax.iota(jnp.int32, LANES)
    base_row = pl.multiple_of(sid * ROWS_A, 8)

    def mx_body(i, m):
        return jnp.maximum(m, jnp.abs(nfb_v[pl.ds(i * LANES, LANES)]))

    @pl.when(sid < NS - 1)
    def _():
        pltpu.sync_copy(zer_hbm, st_sh.at[pl.ds(base_row, ROWS_A)])
        pltpu.sync_copy(nf_hbm.at[pl.ds(base_row, ROWS_A)], nfb_v)
        m = lax.fori_loop(0, ROWS_A // LANES, mx_body,
                          jnp.zeros((LANES,), jnp.float32))
        mrg_v[...] = m

    @pl.when(sid == NS - 1)
    def _():
        pltpu.sync_copy(zer_hbm.at[pl.ds(0, ROWS_LAST)],
                        st_sh.at[pl.ds(base_row, ROWS_LAST)])
        pltpu.sync_copy(nf_hbm.at[pl.ds(base_row, ROWS_LAST)],
                        nfb_v.at[pl.ds(0, ROWS_LAST)])
        m = lax.fori_loop(0, ROWS_LAST // LANES, mx_body,
                          jnp.zeros((LANES,), jnp.float32))
        mrg_v[...] = m

    pltpu.sync_copy(mrg_v, scr_sh.at[sid])
    plsc.subcore_barrier()

    pltpu.sync_copy(scr_sh, scr_v)
    mx = scr_v[0]
    for i in range(1, NS):
        mx = jnp.maximum(mx, scr_v[i])
    for k in (1, 2, 4, 8):
        mrg_v[...] = mx
        mx = jnp.maximum(mx, plsc.load_gather(
            mrg_v, [jnp.bitwise_xor(iota, jnp.int32(k))]))

    A = [par_v[h] for h in range(NUM_HEADS)]
    C = [par_v[NUM_HEADS + h] for h in range(NUM_HEADS)]
    D = [par_v[2 * NUM_HEADS + h] for h in range(NUM_HEADS)]
    E0 = [par_v[3 * NUM_HEADS + h] for h in range(NUM_HEADS)]
    PEN = [par_v[4 * NUM_HEADS + h] for h in range(NUM_HEADS)]
    SH = [jnp.abs(A[h]) * mx + jnp.abs(C[h]) * mx + jnp.abs(D[h])
          + jnp.abs(E0[h]) + jnp.abs(PEN[h]) for h in range(NUM_HEADS)]

    def _fire_loads(par, g):
        pltpu.async_copy(ei_hbm.at[0, g], src_v[par], sem_in[par])
        pltpu.async_copy(ei_hbm.at[1, g], dst_v[par], sem_in[par])
        pltpu.async_copy(cm_hbm.at[g], cm_v[par], sem_in[par])

    def _wait_loads(par, g):
        pltpu.make_async_copy(ei_hbm.at[0, g], src_v[par], sem_in[par]).wait()
        pltpu.make_async_copy(ei_hbm.at[1, g], dst_v[par], sem_in[par]).wait()
        pltpu.make_async_copy(cm_hbm.at[g], cm_v[par], sem_in[par]).wait()

    def _fire_gathers(par):
        pltpu.async_copy(nf_sh.at[src_v[par]], xs_v[par], sem_g[par])
        pltpu.async_copy(nf_sh.at[dst_v[par]], xd_v[par], sem_g[par])

    def _wait_gathers(par):
        pltpu.make_async_copy(nf_sh.at[src_v[par]], xs_v[par],
                              sem_g[par]).wait()
        pltpu.make_async_copy(nf_sh.at[dst_v[par]], xd_v[par],
                              sem_g[par]).wait()

    def _fire_scatters(par):
        pltpu.async_copy(upd_v[par], st_sh.at[dst_v[par]], sem_s[par],
                         add=True)

    def _wait_scatters(par):
        pltpu.make_async_copy(upd_v[par], st_sh.at[dst_v[par]],
                              sem_s[par]).wait()

    def _compute(par):
        if True:
            def cc_body(ccj, cc_carry):
                col = ccj * LANES
                xs = xs_v[par][pl.ds(col, LANES)]
                xd = xd_v[par][pl.ds(col, LANES)]
                cmv = cm_v[par][pl.ds(col, LANES)]
                rowv = col + iota
                for h in range(NUM_HEADS):
                    r = A[h] * xs + C[h] * xd + D[h] * cmv + E0[h]
                    r = jnp.maximum(r, 0.2 * r)
                    ex = jnp.exp(r + cmv * PEN[h] - SH[h])
                    plsc.store_scatter(
                        upd_v[par],
                        [rowv, jnp.full((LANES,), h, jnp.int32)], ex)
                    plsc.store_scatter(
                        upd_v[par],
                        [rowv, jnp.full((LANES,), NUM_HEADS + h, jnp.int32)],
                        xs * ex)
                return cc_carry

            lax.fori_loop(0, EB // LANES, cc_body, 0)

    def blk2_body(u, carry):
        for par in range(2):
            t2 = 2 * u + par
            g = wid + t2 * NW
            gprev = g - NW
            gnext = g + NW

            @pl.when(g < NBLOCKS)
            def _(par=par, g=g):
                _wait_loads(par, g)
                _fire_gathers(par)

            @pl.when(jnp.logical_and(gprev >= 0, gprev < NBLOCKS))
            def _(par=par):
                _wait_scatters(1 - par)

            @pl.when(gnext < NBLOCKS)
            def _(par=par, gnext=gnext):
                _fire_loads(1 - par, gnext)

            @pl.when(g < NBLOCKS)
            def _(par=par):
                _wait_gathers(par)
                _compute(par)
                _fire_scatters(par)
        return carry

    _fire_loads(0, wid)
    lax.fori_loop(0, MAXT // 2, blk2_body, 0)

    g_last = wid + (MAXT - 1) * NW

    @pl.when(g_last < NBLOCKS)
    def _():
        _wait_scatters((MAXT - 1) & 1)

    plsc.subcore_barrier()

    @pl.when(sid < NS - 1)
    def _():
        pltpu.sync_copy(st_sh.at[pl.ds(base_row, ROWS_A)],
                        sts_hbm.at[cid, pl.ds(base_row, ROWS_A)])

    @pl.when(sid == NS - 1)
    def _():
        pltpu.sync_copy(st_sh.at[pl.ds(base_row, ROWS_LAST)],
                        sts_hbm.at[cid, pl.ds(base_row, ROWS_LAST)])


_edge_pass = functools.partial(
    pl.kernel,
    out_type=jax.ShapeDtypeStruct((NC, N_NODES, 2 * NUM_HEADS), jnp.float32),
    mesh=_mesh,
    compiler_params=pltpu.CompilerParams(needs_layout_passes=False,
                                         use_tc_tiling_on_sc=False),
    scratch_types=[
        pltpu.VMEM((5 * NUM_HEADS, LANES), jnp.float32),
    ] + 2 * [
        pltpu.VMEM((EB,), jnp.int32),
        pltpu.VMEM((EB,), jnp.int32),
        pltpu.VMEM((EB,), jnp.float32),
        pltpu.VMEM((EB,), jnp.float32),
        pltpu.VMEM((EB,), jnp.float32),
        pltpu.VMEM((EB, 2 * NUM_HEADS), jnp.float32),
    ] + [
        pltpu.VMEM((ROWS_A,), jnp.float32),
        pltpu.VMEM((LANES,), jnp.float32),
        pltpu.VMEM((NS, LANES), jnp.float32),
        pltpu.VMEM_SHARED((N_NODES,), jnp.float32),
        pltpu.VMEM_SHARED((NS, LANES), jnp.float32),
        pltpu.VMEM_SHARED((N_NODES, 2 * NUM_HEADS), jnp.float32),
    ] + 6 * [pltpu.SemaphoreType.DMA],
)(_edge_pass_body)


_PADN = 100352
_KB = 512
_GRID = _PADN // _KB


def _combine_body(scal_ref, st_ref, out_ref):
    x = st_ref[...]
    s = x[0] + x[1]
    S = s[0:NUM_HEADS]
    T = s[NUM_HEADS:]
    q = jnp.where(S > 0.0, T / S, 0.0)
    out_ref[...] = (jnp.sum(q, axis=0) * scal_ref[0, 0])[None, :]


_combine = pl.pallas_call(
    _combine_body,
    grid=(_GRID,),
    in_specs=[
        pl.BlockSpec(memory_space=pltpu.SMEM),
        pl.BlockSpec((NC, 2 * NUM_HEADS, _KB), lambda i: (0, 0, i)),
    ],
    out_specs=pl.BlockSpec((1, _KB), lambda i: (0, i)),
    out_shape=jax.ShapeDtypeStruct((1, _PADN), jnp.float32),
)


def kernel(node_features, edge_index, cycle_mask, W1, b1, W2, b2,
           cycle_penalty, min_sum_scaler):
    w1 = W1[:, 0]
    a = W2[:, :16] @ w1
    c = W2[:, 16:32] @ w1
    d = W2[:, 32]
    e = b2 + (W2[:, :16] + W2[:, 16:32]) @ b1
    par = jnp.concatenate([a, c, d, e, cycle_penalty])
    par16 = jnp.broadcast_to(par[:, None], (5 * NUM_HEADS, LANES))

    ei = edge_index.reshape(2, NBLOCKS, EB)
    cmr = cycle_mask.reshape(NBLOCKS, EB)
    zer = jnp.zeros((ROWS_A, 2 * NUM_HEADS), jnp.float32)

    sts = _edge_pass(ei, cmr, node_features, par16, zer)

    stt = jnp.pad(jnp.transpose(sts, (0, 2, 1)),
                  ((0, 0), (0, 0), (0, _PADN - N_NODES)))
    scal = (min_sum_scaler * (1.0 / NUM_HEADS)).reshape(1, 1)
    out = _combine(scal, stt)
    return out.reshape(_PADN)[:N_NODES]


# speedup vs baseline: 348.0947x; 1.5879x over previous
import functools

import jax
import jax.numpy as jnp
from jax import lax
from jax.experimental import pallas as pl
from jax.experimental.pallas import tpu as pltpu
from jax.experimental.pallas import tpu_sc as plsc

N_NODES = 100000
N_EDGES = 3200000
NUM_HEADS = 4
NC = 2
NS = 16
NW = NC * NS
LANES = 16
EB = 1024
NBLOCKS = N_EDGES // EB
MAXT = (NBLOCKS + NW - 1) // NW
ROWS_A = 6256
ROWS_LAST = N_NODES - 15 * ROWS_A

_mesh = plsc.VectorSubcoreMesh(
    core_axis_name="c", subcore_axis_name="s", num_cores=NC, num_subcores=NS)


def _edge_pass_body(ei_hbm, cm_hbm, nf_hbm, par_hbm, zer_hbm, sts_hbm,
                    par_v, src_v0, dst_v0, cm_v0, xs_v0, xd_v0, upd_v0,
                    src_v1, dst_v1, cm_v1, xs_v1, xd_v1, upd_v1,
                    nfb_v, mrg_v, scr_v, nf_sh, scr_sh, st_sh,
                    sem_in0, sem_in1, sem_g0, sem_g1, sem_s0, sem_s1):
    src_v = [src_v0, src_v1]
    dst_v = [dst_v0, dst_v1]
    cm_v = [cm_v0, cm_v1]
    xs_v = [xs_v0, xs_v1]
    xd_v = [xd_v0, xd_v1]
    upd_v = [upd_v0, upd_v1]
    sem_in = [sem_in0, sem_in1]
    sem_g = [sem_g0, sem_g1]
    sem_s = [sem_s0, sem_s1]
    cid = lax.axis_index("c")
    sid = lax.axis_index("s")
    wid = sid * NC + cid

    pltpu.sync_copy(par_hbm, par_v)

    @pl.when(sid == 0)
    def _():
        pltpu.sync_copy(nf_hbm, nf_sh)

    iota = lax.iota(jnp.int32, LANES)
    base_row = pl.multiple_of(sid * ROWS_A, 8)

    def mx_body(i, m):
        return jnp.maximum(m, jnp.abs(nfb_v[pl.ds(i * LANES, LANES)]))

    @pl.when(sid < NS - 1)
    def _():
        pltpu.sync_copy(zer_hbm, st_sh.at[pl.ds(base_row, ROWS_A)])
        pltpu.sync_copy(nf_hbm.at[pl.ds(base_row, ROWS_A)], nfb_v)
        m = lax.fori_loop(0, ROWS_A // LANES, mx_body,
                          jnp.zeros((LANES,), jnp.float32))
        mrg_v[...] = m

    @pl.when(sid == NS - 1)
    def _():
        pltpu.sync_copy(zer_hbm.at[pl.ds(0, ROWS_LAST)],
                        st_sh.at[pl.ds(base_row, ROWS_LAST)])
        pltpu.sync_copy(nf_hbm.at[pl.ds(base_row, ROWS_LAST)],
                        nfb_v.at[pl.ds(0, ROWS_LAST)])
        m = lax.fori_loop(0, ROWS_LAST // LANES, mx_body,
                          jnp.zeros((LANES,), jnp.float32))
        mrg_v[...] = m

    pltpu.sync_copy(mrg_v, scr_sh.at[sid])
    plsc.subcore_barrier()

    pltpu.sync_copy(scr_sh, scr_v)
    mx = scr_v[0]
    for i in range(1, NS):
        mx = jnp.maximum(mx, scr_v[i])
    for k in (1, 2, 4, 8):
        mrg_v[...] = mx
        mx = jnp.maximum(mx, plsc.load_gather(
            mrg_v, [jnp.bitwise_xor(iota, jnp.int32(k))]))

    A = [par_v[h] for h in range(NUM_HEADS)]
    C = [par_v[NUM_HEADS + h] for h in range(NUM_HEADS)]
    D = [par_v[2 * NUM_HEADS + h] for h in range(NUM_HEADS)]
    E0 = [par_v[3 * NUM_HEADS + h] for h in range(NUM_HEADS)]
    PEN = [par_v[4 * NUM_HEADS + h] for h in range(NUM_HEADS)]
    SH = [jnp.abs(A[h]) * mx + jnp.abs(C[h]) * mx + jnp.abs(D[h])
          + jnp.abs(E0[h]) + jnp.abs(PEN[h]) for h in range(NUM_HEADS)]

    def _fire_loads(par, g):
        pltpu.async_copy(ei_hbm.at[0, g], src_v[par], sem_in[par])
        pltpu.async_copy(ei_hbm.at[1, g], dst_v[par], sem_in[par])
        pltpu.async_copy(cm_hbm.at[g], cm_v[par], sem_in[par])

    def _wait_loads(par, g):
        pltpu.make_async_copy(ei_hbm.at[0, g], src_v[par], sem_in[par]).wait()
        pltpu.make_async_copy(ei_hbm.at[1, g], dst_v[par], sem_in[par]).wait()
        pltpu.make_async_copy(cm_hbm.at[g], cm_v[par], sem_in[par]).wait()

    def _fire_gathers(par):
        pltpu.async_copy(nf_sh.at[src_v[par]], xs_v[par], sem_g[par])
        pltpu.async_copy(nf_sh.at[dst_v[par]], xd_v[par], sem_g[par])

    def _wait_gathers(par):
        pltpu.make_async_copy(nf_sh.at[src_v[par]], xs_v[par],
                              sem_g[par]).wait()
        pltpu.make_async_copy(nf_sh.at[dst_v[par]], xd_v[par],
                              sem_g[par]).wait()

    def _fire_scatters(par):
        pltpu.async_copy(upd_v[par], st_sh.at[dst_v[par]], sem_s[par],
                         add=True)

    def _wait_scatters(par):
        pltpu.make_async_copy(upd_v[par], st_sh.at[dst_v[par]],
                              sem_s[par]).wait()

    def _compute(par):
        if True:
            def cc_body(ccj, cc_carry):
                col = ccj * LANES
                xs = xs_v[par][pl.ds(col, LANES)]
                xd = xd_v[par][pl.ds(col, LANES)]
                cmv = cm_v[par][pl.ds(col, LANES)]
                rowv = col + iota
                for h in range(NUM_HEADS):
                    r = A[h] * xs + C[h] * xd + D[h] * cmv + E0[h]
                    r = jnp.maximum(r, 0.2 * r)
                    ex = jnp.exp(r + cmv * PEN[h] - SH[h])
                    plsc.store_scatter(
                        upd_v[par],
                        [rowv, jnp.full((LANES,), h, jnp.int32)], ex)
                    plsc.store_scatter(
                        upd_v[par],
                        [rowv, jnp.full((LANES,), NUM_HEADS + h, jnp.int32)],
                        xs * ex)
                return cc_carry

            lax.fori_loop(0, EB // LANES, cc_body, 0)

    def blk2_body(u, carry):
        for par in range(2):
            t2 = 2 * u + par
            g = wid + t2 * NW
            gprev = g - NW
            gnext = g + NW

            @pl.when(g < NBLOCKS)
            def _(par=par, g=g):
                _wait_loads(par, g)
                _fire_gathers(par)

            @pl.when(jnp.logical_and(gprev >= 0, gprev < NBLOCKS))
            def _(par=par):
                _wait_scatters(1 - par)

            @pl.when(gnext < NBLOCKS)
            def _(par=par, gnext=gnext):
                _fire_loads(1 - par, gnext)

            @pl.when(g < NBLOCKS)
            def _(par=par):
                _wait_gathers(par)
                _compute(par)
                _fire_scatters(par)
        return carry

    _fire_loads(0, wid)
    lax.fori_loop(0, MAXT // 2, blk2_body, 0)

    g_last = wid + (MAXT - 1) * NW

    @pl.when(g_last < NBLOCKS)
    def _():
        _wait_scatters((MAXT - 1) & 1)

    plsc.subcore_barrier()

    @pl.when(sid < NS - 1)
    def _():
        pltpu.sync_copy(st_sh.at[pl.ds(base_row, ROWS_A)],
                        sts_hbm.at[cid, pl.ds(base_row, ROWS_A)])

    @pl.when(sid == NS - 1)
    def _():
        pltpu.sync_copy(st_sh.at[pl.ds(base_row, ROWS_LAST)],
                        sts_hbm.at[cid, pl.ds(base_row, ROWS_LAST)])


_edge_pass = functools.partial(
    pl.kernel,
    out_type=jax.ShapeDtypeStruct((NC, N_NODES, 2 * NUM_HEADS), jnp.float32),
    mesh=_mesh,
    compiler_params=pltpu.CompilerParams(needs_layout_passes=False,
                                         use_tc_tiling_on_sc=False),
    scratch_types=[
        pltpu.VMEM((5 * NUM_HEADS, LANES), jnp.float32),
    ] + 2 * [
        pltpu.VMEM((EB,), jnp.int32),
        pltpu.VMEM((EB,), jnp.int32),
        pltpu.VMEM((EB,), jnp.float32),
        pltpu.VMEM((EB,), jnp.float32),
        pltpu.VMEM((EB,), jnp.float32),
        pltpu.VMEM((EB, 2 * NUM_HEADS), jnp.float32),
    ] + [
        pltpu.VMEM((ROWS_A,), jnp.float32),
        pltpu.VMEM((LANES,), jnp.float32),
        pltpu.VMEM((NS, LANES), jnp.float32),
        pltpu.VMEM_SHARED((N_NODES,), jnp.float32),
        pltpu.VMEM_SHARED((NS, LANES), jnp.float32),
        pltpu.VMEM_SHARED((N_NODES, 2 * NUM_HEADS), jnp.float32),
    ] + 6 * [pltpu.SemaphoreType.DMA],
)(_edge_pass_body)


CROWS = 3136
CROWS_LAST = N_NODES - (NW - 1) * CROWS


def _combine_body(sts_hbm, scl_hbm, out_hbm, va, vb, vo, scl_v):
    cid = lax.axis_index("c")
    sid = lax.axis_index("s")
    w = sid * NC + cid
    base = pl.multiple_of(w * CROWS, 8)
    pltpu.sync_copy(scl_hbm, scl_v)
    scl = scl_v[0]
    iota = lax.iota(jnp.int32, LANES)

    def _do(R):
        pltpu.sync_copy(sts_hbm.at[0, pl.ds(base, R)], va.at[pl.ds(0, R)])
        pltpu.sync_copy(sts_hbm.at[1, pl.ds(base, R)], vb.at[pl.ds(0, R)])

        def grp_body(grp, carry):
            rows = grp * LANES + iota
            q = jnp.zeros((LANES,), jnp.float32)
            for h in range(NUM_HEADS):
                ch = jnp.full((LANES,), h, jnp.int32)
                ct = jnp.full((LANES,), NUM_HEADS + h, jnp.int32)
                S = (plsc.load_gather(va, [rows, ch])
                     + plsc.load_gather(vb, [rows, ch]))
                T = (plsc.load_gather(va, [rows, ct])
                     + plsc.load_gather(vb, [rows, ct]))
                q = q + jnp.where(S > 0.0, T / S, 0.0)
            vo[pl.ds(grp * LANES, LANES)] = q * scl
            return carry

        lax.fori_loop(0, R // LANES, grp_body, 0)
        pltpu.sync_copy(vo.at[pl.ds(0, R)], out_hbm.at[pl.ds(base, R)])

    @pl.when(w < NW - 1)
    def _():
        _do(CROWS)

    @pl.when(w == NW - 1)
    def _():
        _do(CROWS_LAST)


_combine = functools.partial(
    pl.kernel,
    out_type=jax.ShapeDtypeStruct((N_NODES,), jnp.float32),
    mesh=_mesh,
    compiler_params=pltpu.CompilerParams(needs_layout_passes=False,
                                         use_tc_tiling_on_sc=False),
    scratch_types=[
        pltpu.VMEM((CROWS, 2 * NUM_HEADS), jnp.float32),
        pltpu.VMEM((CROWS, 2 * NUM_HEADS), jnp.float32),
        pltpu.VMEM((CROWS,), jnp.float32),
        pltpu.VMEM((1, LANES), jnp.float32),
    ],
)(_combine_body)


def kernel(node_features, edge_index, cycle_mask, W1, b1, W2, b2,
           cycle_penalty, min_sum_scaler):
    w1 = W1[:, 0]
    a = W2[:, :16] @ w1
    c = W2[:, 16:32] @ w1
    d = W2[:, 32]
    e = b2 + (W2[:, :16] + W2[:, 16:32]) @ b1
    par = jnp.concatenate([a, c, d, e, cycle_penalty])
    par16 = jnp.broadcast_to(par[:, None], (5 * NUM_HEADS, LANES))

    ei = edge_index.reshape(2, NBLOCKS, EB)
    cmr = cycle_mask.reshape(NBLOCKS, EB)
    zer = jnp.zeros((ROWS_A, 2 * NUM_HEADS), jnp.float32)

    sts = _edge_pass(ei, cmr, node_features, par16, zer)

    scl16 = jnp.broadcast_to(
        (min_sum_scaler * (1.0 / NUM_HEADS)).reshape(1, 1), (1, LANES))
    return _combine(sts, scl16)


# speedup vs baseline: 349.3572x; 1.0036x over previous
import functools

import jax
import jax.numpy as jnp
from jax import lax
from jax.experimental import pallas as pl
from jax.experimental.pallas import tpu as pltpu
from jax.experimental.pallas import tpu_sc as plsc

N_NODES = 100000
N_EDGES = 3200000
NUM_HEADS = 4
NC = 2
NS = 16
NW = NC * NS
LANES = 16
EB = 1024
NBLOCKS = N_EDGES // EB
MAXT = (NBLOCKS + NW - 1) // NW
ROWS_A = 6256
ROWS_LAST = N_NODES - 15 * ROWS_A

_mesh = plsc.VectorSubcoreMesh(
    core_axis_name="c", subcore_axis_name="s", num_cores=NC, num_subcores=NS)


def _edge_pass_body(ei_hbm, cm_hbm, nf_hbm, par_hbm, zer_hbm, sts_hbm,
                    par_v, src_v0, dst_v0, cm_v0, xs_v0, xd_v0, upd_v0,
                    src_v1, dst_v1, cm_v1, xs_v1, xd_v1, upd_v1,
                    sdst_v0, sdst_v1,
                    nfb_v, mrg_v, scr_v, nf_sh, scr_sh, st_sh,
                    sem_in0, sem_in1, sem_g0, sem_g1, sem_s0, sem_s1):
    src_v = [src_v0, src_v1]
    dst_v = [dst_v0, dst_v1]
    cm_v = [cm_v0, cm_v1]
    xs_v = [xs_v0, xs_v1]
    xd_v = [xd_v0, xd_v1]
    upd_v = [upd_v0, upd_v1]
    sdst_v = [sdst_v0, sdst_v1]
    sem_in = [sem_in0, sem_in1]
    sem_g = [sem_g0, sem_g1]
    sem_s = [sem_s0, sem_s1]
    cid = lax.axis_index("c")
    sid = lax.axis_index("s")
    wid = sid * NC + cid

    pltpu.sync_copy(par_hbm, par_v)

    @pl.when(sid == 0)
    def _():
        pltpu.sync_copy(nf_hbm, nf_sh)

    iota = lax.iota(jnp.int32, LANES)
    base_row = pl.multiple_of(sid * ROWS_A, 8)

    def mx_body(i, m):
        return jnp.maximum(m, jnp.abs(nfb_v[pl.ds(i * LANES, LANES)]))

    @pl.when(sid < NS - 1)
    def _():
        pltpu.sync_copy(zer_hbm, st_sh.at[pl.ds(base_row, ROWS_A)])
        pltpu.sync_copy(nf_hbm.at[pl.ds(base_row, ROWS_A)], nfb_v)
        m = lax.fori_loop(0, ROWS_A // LANES, mx_body,
                          jnp.zeros((LANES,), jnp.float32))
        mrg_v[...] = m

    @pl.when(sid == NS - 1)
    def _():
        pltpu.sync_copy(zer_hbm.at[pl.ds(0, ROWS_LAST)],
                        st_sh.at[pl.ds(base_row, ROWS_LAST)])
        pltpu.sync_copy(nf_hbm.at[pl.ds(base_row, ROWS_LAST)],
                        nfb_v.at[pl.ds(0, ROWS_LAST)])
        m = lax.fori_loop(0, ROWS_LAST // LANES, mx_body,
                          jnp.zeros((LANES,), jnp.float32))
        mrg_v[...] = m

    pltpu.sync_copy(mrg_v, scr_sh.at[sid])
    plsc.subcore_barrier()

    pltpu.sync_copy(scr_sh, scr_v)
    mx = scr_v[0]
    for i in range(1, NS):
        mx = jnp.maximum(mx, scr_v[i])
    for k in (1, 2, 4, 8):
        mrg_v[...] = mx
        mx = jnp.maximum(mx, plsc.load_gather(
            mrg_v, [jnp.bitwise_xor(iota, jnp.int32(k))]))

    A = [par_v[h] for h in range(NUM_HEADS)]
    C = [par_v[NUM_HEADS + h] for h in range(NUM_HEADS)]
    D = [par_v[2 * NUM_HEADS + h] for h in range(NUM_HEADS)]
    E0 = [par_v[3 * NUM_HEADS + h] for h in range(NUM_HEADS)]
    PEN = [par_v[4 * NUM_HEADS + h] for h in range(NUM_HEADS)]
    SH = [jnp.abs(A[h]) * mx + jnp.abs(C[h]) * mx + jnp.abs(D[h])
          + jnp.abs(E0[h]) + jnp.abs(PEN[h]) for h in range(NUM_HEADS)]

    def _fire_loads(par, g):
        pltpu.async_copy(ei_hbm.at[0, g], src_v[par], sem_in[par])
        pltpu.async_copy(ei_hbm.at[1, g], dst_v[par], sem_in[par])
        pltpu.async_copy(cm_hbm.at[g], cm_v[par], sem_in[par])

    def _wait_loads(par, g):
        pltpu.make_async_copy(ei_hbm.at[0, g], src_v[par], sem_in[par]).wait()
        pltpu.make_async_copy(ei_hbm.at[1, g], dst_v[par], sem_in[par]).wait()
        pltpu.make_async_copy(cm_hbm.at[g], cm_v[par], sem_in[par]).wait()

    def _fire_gathers(par):
        pltpu.async_copy(nf_sh.at[src_v[par]], xs_v[par], sem_g[par])
        pltpu.async_copy(nf_sh.at[dst_v[par]], xd_v[par], sem_g[par])

    def _wait_gathers(par):
        pltpu.make_async_copy(nf_sh.at[src_v[par]], xs_v[par],
                              sem_g[par]).wait()
        pltpu.make_async_copy(nf_sh.at[dst_v[par]], xd_v[par],
                              sem_g[par]).wait()

    def _fire_scatters(par):
        pltpu.async_copy(upd_v[par], st_sh.at[sdst_v[par]], sem_s[par],
                         add=True)

    def _wait_scatters(par):
        pltpu.make_async_copy(upd_v[par], st_sh.at[sdst_v[par]],
                              sem_s[par]).wait()

    def _compute(par):
        if True:
            def cc_body(ccj, cc_carry):
                col = ccj * LANES
                xs = xs_v[par][pl.ds(col, LANES)]
                xd = xd_v[par][pl.ds(col, LANES)]
                cmv = cm_v[par][pl.ds(col, LANES)]
                sdst_v[par][pl.ds(col, LANES)] = dst_v[par][pl.ds(col, LANES)]
                rowv = col + iota
                for h in range(NUM_HEADS):
                    r = A[h] * xs + C[h] * xd + D[h] * cmv + E0[h]
                    r = jnp.maximum(r, 0.2 * r)
                    ex = jnp.exp(r + cmv * PEN[h] - SH[h])
                    plsc.store_scatter(
                        upd_v[par],
                        [rowv, jnp.full((LANES,), h, jnp.int32)], ex)
                    plsc.store_scatter(
                        upd_v[par],
                        [rowv, jnp.full((LANES,), NUM_HEADS + h, jnp.int32)],
                        xs * ex)
                return cc_carry

            lax.fori_loop(0, EB // LANES, cc_body, 0)

    def blk2_body(u, carry):
        for par in range(2):
            t2 = 2 * u + par
            g = wid + t2 * NW
            gprev = g - NW
            gnext = g + NW

            @pl.when(g < NBLOCKS)
            def _(par=par, g=g):
                _wait_loads(par, g)
                _fire_gathers(par)

            @pl.when(gnext < NBLOCKS)
            def _(par=par, gnext=gnext):
                _fire_loads(1 - par, gnext)

            @pl.when(g < NBLOCKS)
            def _(par=par):
                _wait_gathers(par)
                _compute(par)

            @pl.when(jnp.logical_and(gprev >= 0, gprev < NBLOCKS))
            def _(par=par):
                _wait_scatters(1 - par)

            @pl.when(g < NBLOCKS)
            def _(par=par):
                _fire_scatters(par)
        return carry

    _fire_loads(0, wid)
    lax.fori_loop(0, MAXT // 2, blk2_body, 0)

    g_last = wid + (MAXT - 1) * NW

    @pl.when(g_last < NBLOCKS)
    def _():
        _wait_scatters((MAXT - 1) & 1)

    plsc.subcore_barrier()

    @pl.when(sid < NS - 1)
    def _():
        pltpu.sync_copy(st_sh.at[pl.ds(base_row, ROWS_A)],
                        sts_hbm.at[cid, pl.ds(base_row, ROWS_A)])

    @pl.when(sid == NS - 1)
    def _():
        pltpu.sync_copy(st_sh.at[pl.ds(base_row, ROWS_LAST)],
                        sts_hbm.at[cid, pl.ds(base_row, ROWS_LAST)])


_edge_pass = functools.partial(
    pl.kernel,
    out_type=jax.ShapeDtypeStruct((NC, N_NODES, 2 * NUM_HEADS), jnp.float32),
    mesh=_mesh,
    compiler_params=pltpu.CompilerParams(needs_layout_passes=False,
                                         use_tc_tiling_on_sc=False),
    scratch_types=[
        pltpu.VMEM((5 * NUM_HEADS, LANES), jnp.float32),
    ] + 2 * [
        pltpu.VMEM((EB,), jnp.int32),
        pltpu.VMEM((EB,), jnp.int32),
        pltpu.VMEM((EB,), jnp.float32),
        pltpu.VMEM((EB,), jnp.float32),
        pltpu.VMEM((EB,), jnp.float32),
        pltpu.VMEM((EB, 2 * NUM_HEADS), jnp.float32),
    ] + 2 * [
        pltpu.VMEM((EB,), jnp.int32),
    ] + [
        pltpu.VMEM((ROWS_A,), jnp.float32),
        pltpu.VMEM((LANES,), jnp.float32),
        pltpu.VMEM((NS, LANES), jnp.float32),
        pltpu.VMEM_SHARED((N_NODES,), jnp.float32),
        pltpu.VMEM_SHARED((NS, LANES), jnp.float32),
        pltpu.VMEM_SHARED((N_NODES, 2 * NUM_HEADS), jnp.float32),
    ] + 6 * [pltpu.SemaphoreType.DMA],
)(_edge_pass_body)


CROWS = 3136
CROWS_LAST = N_NODES - (NW - 1) * CROWS


def _combine_body(sts_hbm, scl_hbm, out_hbm, va, vb, vo, scl_v):
    cid = lax.axis_index("c")
    sid = lax.axis_index("s")
    w = sid * NC + cid
    base = pl.multiple_of(w * CROWS, 8)
    pltpu.sync_copy(scl_hbm, scl_v)
    scl = scl_v[0]
    iota = lax.iota(jnp.int32, LANES)

    def _do(R):
        pltpu.sync_copy(sts_hbm.at[0, pl.ds(base, R)], va.at[pl.ds(0, R)])
        pltpu.sync_copy(sts_hbm.at[1, pl.ds(base, R)], vb.at[pl.ds(0, R)])

        def grp_body(grp, carry):
            rows = grp * LANES + iota
            q = jnp.zeros((LANES,), jnp.float32)
            for h in range(NUM_HEADS):
                ch = jnp.full((LANES,), h, jnp.int32)
                ct = jnp.full((LANES,), NUM_HEADS + h, jnp.int32)
                S = (plsc.load_gather(va, [rows, ch])
                     + plsc.load_gather(vb, [rows, ch]))
                T = (plsc.load_gather(va, [rows, ct])
                     + plsc.load_gather(vb, [rows, ct]))
                q = q + jnp.where(S > 0.0, T / S, 0.0)
            vo[pl.ds(grp * LANES, LANES)] = q * scl
            return carry

        lax.fori_loop(0, R // LANES, grp_body, 0)
        pltpu.sync_copy(vo.at[pl.ds(0, R)], out_hbm.at[pl.ds(base, R)])

    @pl.when(w < NW - 1)
    def _():
        _do(CROWS)

    @pl.when(w == NW - 1)
    def _():
        _do(CROWS_LAST)


_combine = functools.partial(
    pl.kernel,
    out_type=jax.ShapeDtypeStruct((N_NODES,), jnp.float32),
    mesh=_mesh,
    compiler_params=pltpu.CompilerParams(needs_layout_passes=False,
                                         use_tc_tiling_on_sc=False),
    scratch_types=[
        pltpu.VMEM((CROWS, 2 * NUM_HEADS), jnp.float32),
        pltpu.VMEM((CROWS, 2 * NUM_HEADS), jnp.float32),
        pltpu.VMEM((CROWS,), jnp.float32),
        pltpu.VMEM((1, LANES), jnp.float32),
    ],
)(_combine_body)


def kernel(node_features, edge_index, cycle_mask, W1, b1, W2, b2,
           cycle_penalty, min_sum_scaler):
    w1 = W1[:, 0]
    a = W2[:, :16] @ w1
    c = W2[:, 16:32] @ w1
    d = W2[:, 32]
    e = b2 + (W2[:, :16] + W2[:, 16:32]) @ b1
    par = jnp.concatenate([a, c, d, e, cycle_penalty])
    par16 = jnp.broadcast_to(par[:, None], (5 * NUM_HEADS, LANES))

    ei = edge_index.reshape(2, NBLOCKS, EB)
    cmr = cycle_mask.reshape(NBLOCKS, EB)
    zer = jnp.zeros((ROWS_A, 2 * NUM_HEADS), jnp.float32)

    sts = _edge_pass(ei, cmr, node_features, par16, zer)

    scl16 = jnp.broadcast_to(
        (min_sum_scaler * (1.0 / NUM_HEADS)).reshape(1, 1), (1, LANES))
    return _combine(sts, scl16)


# speedup vs baseline: 363.9668x; 1.0418x over previous
import functools

import jax
import jax.numpy as jnp
from jax import lax
from jax.experimental import pallas as pl
from jax.experimental.pallas import tpu as pltpu
from jax.experimental.pallas import tpu_sc as plsc

N_NODES = 100000
N_EDGES = 3200000
NUM_HEADS = 4
NC = 2
NS = 16
NW = NC * NS
LANES = 16
EB = 1024
NBLOCKS = N_EDGES // EB
MAXT = (NBLOCKS + NW - 1) // NW
ROWS_A = 6256
ROWS_LAST = N_NODES - 15 * ROWS_A
PKROWS = 3136
PKROWS_LAST = N_NODES // 2 - 15 * PKROWS

_mesh = plsc.VectorSubcoreMesh(
    core_axis_name="c", subcore_axis_name="s", num_cores=NC, num_subcores=NS)


def _edge_pass_body(ei_hbm, cm_hbm, pk_hbm, par_hbm, zer_hbm, sts_hbm,
                    par_v, src_v0, dst_v0, cm_v0, upd_v0,
                    src_v1, dst_v1, cm_v1, upd_v1,
                    sdst_v0, sdst_v1,
                    pk_v, mrg_v, scr_v, scr_sh, st_sh,
                    sem_in0, sem_in1, sem_s0, sem_s1):
    src_v = [src_v0, src_v1]
    dst_v = [dst_v0, dst_v1]
    cm_v = [cm_v0, cm_v1]
    upd_v = [upd_v0, upd_v1]
    sdst_v = [sdst_v0, sdst_v1]
    sem_in = [sem_in0, sem_in1]
    sem_s = [sem_s0, sem_s1]
    cid = lax.axis_index("c")
    sid = lax.axis_index("s")
    wid = sid * NC + cid

    pltpu.sync_copy(par_hbm, par_v)
    pltpu.sync_copy(pk_hbm, pk_v)

    iota = lax.iota(jnp.int32, LANES)
    base_row = pl.multiple_of(sid * ROWS_A, 8)

    def mx_body(i, m, base=0):
        w = pk_v[pl.ds(base + i * LANES, LANES)]
        lo = plsc.bitcast(jnp.left_shift(w, 16), jnp.float32)
        hi = plsc.bitcast(jnp.bitwise_and(w, jnp.int32(-65536)), jnp.float32)
        return jnp.maximum(m, jnp.maximum(jnp.abs(lo), jnp.abs(hi)))

    @pl.when(sid < NS - 1)
    def _():
        pltpu.sync_copy(zer_hbm, st_sh.at[pl.ds(base_row, ROWS_A)])
        m = lax.fori_loop(
            0, PKROWS // LANES,
            functools.partial(mx_body, base=sid * PKROWS),
            jnp.zeros((LANES,), jnp.float32))
        mrg_v[...] = m

    @pl.when(sid == NS - 1)
    def _():
        pltpu.sync_copy(zer_hbm.at[pl.ds(0, ROWS_LAST)],
                        st_sh.at[pl.ds(base_row, ROWS_LAST)])
        m = lax.fori_loop(
            0, PKROWS_LAST // LANES,
            functools.partial(mx_body, base=(NS - 1) * PKROWS),
            jnp.zeros((LANES,), jnp.float32))
        mrg_v[...] = m

    pltpu.sync_copy(mrg_v, scr_sh.at[sid])
    plsc.subcore_barrier()

    pltpu.sync_copy(scr_sh, scr_v)
    mx = scr_v[0]
    for i in range(1, NS):
        mx = jnp.maximum(mx, scr_v[i])
    for k in (1, 2, 4, 8):
        mrg_v[...] = mx
        mx = jnp.maximum(mx, plsc.load_gather(
            mrg_v, [jnp.bitwise_xor(iota, jnp.int32(k))]))

    A = [par_v[h] for h in range(NUM_HEADS)]
    C = [par_v[NUM_HEADS + h] for h in range(NUM_HEADS)]
    D = [par_v[2 * NUM_HEADS + h] for h in range(NUM_HEADS)]
    E0 = [par_v[3 * NUM_HEADS + h] for h in range(NUM_HEADS)]
    PEN = [par_v[4 * NUM_HEADS + h] for h in range(NUM_HEADS)]
    SH = [jnp.abs(A[h]) * mx + jnp.abs(C[h]) * mx + jnp.abs(D[h])
          + jnp.abs(E0[h]) + jnp.abs(PEN[h]) for h in range(NUM_HEADS)]

    def _fire_loads(par, g):
        pltpu.async_copy(ei_hbm.at[0, g], src_v[par], sem_in[par])
        pltpu.async_copy(ei_hbm.at[1, g], dst_v[par], sem_in[par])
        pltpu.async_copy(cm_hbm.at[g], cm_v[par], sem_in[par])

    def _wait_loads(par, g):
        pltpu.make_async_copy(ei_hbm.at[0, g], src_v[par], sem_in[par]).wait()
        pltpu.make_async_copy(ei_hbm.at[1, g], dst_v[par], sem_in[par]).wait()
        pltpu.make_async_copy(cm_hbm.at[g], cm_v[par], sem_in[par]).wait()

    def _fire_scatters(par):
        pltpu.async_copy(upd_v[par], st_sh.at[sdst_v[par]], sem_s[par],
                         add=True)

    def _wait_scatters(par):
        pltpu.make_async_copy(upd_v[par], st_sh.at[sdst_v[par]],
                              sem_s[par]).wait()

    def _compute(par):
        if True:
            def cc_body(ccj, cc_carry):
                col = ccj * LANES
                si = src_v[par][pl.ds(col, LANES)]
                di = dst_v[par][pl.ds(col, LANES)]
                cmv = cm_v[par][pl.ds(col, LANES)]
                sdst_v[par][pl.ds(col, LANES)] = di

                def nf_at(idx):
                    w = plsc.load_gather(pk_v, [jnp.right_shift(idx, 1)])
                    lo = plsc.bitcast(jnp.left_shift(w, 16), jnp.float32)
                    hi = plsc.bitcast(
                        jnp.bitwise_and(w, jnp.int32(-65536)), jnp.float32)
                    return jnp.where(
                        jnp.bitwise_and(idx, 1) == 1, hi, lo)

                xs = nf_at(si)
                xd = nf_at(di)
                rowv = col + iota
                for h in range(NUM_HEADS):
                    r = A[h] * xs + C[h] * xd + D[h] * cmv + E0[h]
                    r = jnp.maximum(r, 0.2 * r)
                    ex = jnp.exp(r + cmv * PEN[h] - SH[h])
                    plsc.store_scatter(
                        upd_v[par],
                        [rowv, jnp.full((LANES,), h, jnp.int32)], ex)
                    plsc.store_scatter(
                        upd_v[par],
                        [rowv, jnp.full((LANES,), NUM_HEADS + h, jnp.int32)],
                        xs * ex)
                return cc_carry

            lax.fori_loop(0, EB // LANES, cc_body, 0)

    def blk2_body(u, carry):
        for par in range(2):
            t2 = 2 * u + par
            g = wid + t2 * NW
            gprev = g - NW
            gnext = g + NW

            @pl.when(g < NBLOCKS)
            def _(par=par, g=g):
                _wait_loads(par, g)

            @pl.when(gnext < NBLOCKS)
            def _(par=par, gnext=gnext):
                _fire_loads(1 - par, gnext)

            @pl.when(g < NBLOCKS)
            def _(par=par):
                _compute(par)

            @pl.when(jnp.logical_and(gprev >= 0, gprev < NBLOCKS))
            def _(par=par):
                _wait_scatters(1 - par)

            @pl.when(g < NBLOCKS)
            def _(par=par):
                _fire_scatters(par)
        return carry

    _fire_loads(0, wid)
    lax.fori_loop(0, MAXT // 2, blk2_body, 0)

    g_last = wid + (MAXT - 1) * NW

    @pl.when(g_last < NBLOCKS)
    def _():
        _wait_scatters((MAXT - 1) & 1)

    plsc.subcore_barrier()

    @pl.when(sid < NS - 1)
    def _():
        pltpu.sync_copy(st_sh.at[pl.ds(base_row, ROWS_A)],
                        sts_hbm.at[cid, pl.ds(base_row, ROWS_A)])

    @pl.when(sid == NS - 1)
    def _():
        pltpu.sync_copy(st_sh.at[pl.ds(base_row, ROWS_LAST)],
                        sts_hbm.at[cid, pl.ds(base_row, ROWS_LAST)])


_edge_pass = functools.partial(
    pl.kernel,
    out_type=jax.ShapeDtypeStruct((NC, N_NODES, 2 * NUM_HEADS), jnp.float32),
    mesh=_mesh,
    compiler_params=pltpu.CompilerParams(needs_layout_passes=False,
                                         use_tc_tiling_on_sc=False),
    scratch_types=[
        pltpu.VMEM((5 * NUM_HEADS, LANES), jnp.float32),
    ] + 2 * [
        pltpu.VMEM((EB,), jnp.int32),
        pltpu.VMEM((EB,), jnp.int32),
        pltpu.VMEM((EB,), jnp.float32),
        pltpu.VMEM((EB, 2 * NUM_HEADS), jnp.float32),
    ] + 2 * [
        pltpu.VMEM((EB,), jnp.int32),
    ] + [
        pltpu.VMEM((N_NODES // 2,), jnp.int32),
        pltpu.VMEM((LANES,), jnp.float32),
        pltpu.VMEM((NS, LANES), jnp.float32),
        pltpu.VMEM_SHARED((NS, LANES), jnp.float32),
        pltpu.VMEM_SHARED((N_NODES, 2 * NUM_HEADS), jnp.float32),
    ] + 4 * [pltpu.SemaphoreType.DMA],
)(_edge_pass_body)


CROWS = 3136
CROWS_LAST = N_NODES - (NW - 1) * CROWS


def _combine_body(sts_hbm, scl_hbm, out_hbm, va, vb, vo, scl_v):
    cid = lax.axis_index("c")
    sid = lax.axis_index("s")
    w = sid * NC + cid
    base = pl.multiple_of(w * CROWS, 8)
    pltpu.sync_copy(scl_hbm, scl_v)
    scl = scl_v[0]
    iota = lax.iota(jnp.int32, LANES)

    def _do(R):
        pltpu.sync_copy(sts_hbm.at[0, pl.ds(base, R)], va.at[pl.ds(0, R)])
        pltpu.sync_copy(sts_hbm.at[1, pl.ds(base, R)], vb.at[pl.ds(0, R)])

        def grp_body(grp, carry):
            rows = grp * LANES + iota
            q = jnp.zeros((LANES,), jnp.float32)
            for h in range(NUM_HEADS):
                ch = jnp.full((LANES,), h, jnp.int32)
                ct = jnp.full((LANES,), NUM_HEADS + h, jnp.int32)
                S = (plsc.load_gather(va, [rows, ch])
                     + plsc.load_gather(vb, [rows, ch]))
                T = (plsc.load_gather(va, [rows, ct])
                     + plsc.load_gather(vb, [rows, ct]))
                q = q + jnp.where(S > 0.0, T / S, 0.0)
            vo[pl.ds(grp * LANES, LANES)] = q * scl
            return carry

        lax.fori_loop(0, R // LANES, grp_body, 0)
        pltpu.sync_copy(vo.at[pl.ds(0, R)], out_hbm.at[pl.ds(base, R)])

    @pl.when(w < NW - 1)
    def _():
        _do(CROWS)

    @pl.when(w == NW - 1)
    def _():
        _do(CROWS_LAST)


_combine = functools.partial(
    pl.kernel,
    out_type=jax.ShapeDtypeStruct((N_NODES,), jnp.float32),
    mesh=_mesh,
    compiler_params=pltpu.CompilerParams(needs_layout_passes=False,
                                         use_tc_tiling_on_sc=False),
    scratch_types=[
        pltpu.VMEM((CROWS, 2 * NUM_HEADS), jnp.float32),
        pltpu.VMEM((CROWS, 2 * NUM_HEADS), jnp.float32),
        pltpu.VMEM((CROWS,), jnp.float32),
        pltpu.VMEM((1, LANES), jnp.float32),
    ],
)(_combine_body)


def kernel(node_features, edge_index, cycle_mask, W1, b1, W2, b2,
           cycle_penalty, min_sum_scaler):
    w1 = W1[:, 0]
    a = W2[:, :16] @ w1
    c = W2[:, 16:32] @ w1
    d = W2[:, 32]
    e = b2 + (W2[:, :16] + W2[:, 16:32]) @ b1
    par = jnp.concatenate([a, c, d, e, cycle_penalty])
    par16 = jnp.broadcast_to(par[:, None], (5 * NUM_HEADS, LANES))

    ei = edge_index.reshape(2, NBLOCKS, EB)
    cmr = cycle_mask.reshape(NBLOCKS, EB)
    zer = jnp.zeros((ROWS_A, 2 * NUM_HEADS), jnp.float32)

    pk = jax.lax.bitcast_convert_type(
        node_features.astype(jnp.bfloat16).reshape(N_NODES // 2, 2),
        jnp.int32)
    sts = _edge_pass(ei, cmr, pk, par16, zer)

    scl16 = jnp.broadcast_to(
        (min_sum_scaler * (1.0 / NUM_HEADS)).reshape(1, 1), (1, LANES))
    return _combine(sts, scl16)
